# baseline (speedup 1.0000x reference)
import functools

import jax
import jax.numpy as jnp
import numpy as np
from jax import lax
from jax.experimental import pallas as pl
from jax.experimental.pallas import tpu as pltpu
from jax.experimental.pallas import tpu_sc as plsc

NUM_ROWS = 1_000_000
DIM = 64
HIST = 50
BATCH = 4096
QSCALE = 255.0


_CB = 65536
_NBLK = (NUM_ROWS + _CB - 1) // _CB
_CP = _CB // 8


_CB1 = 65536
_NBLK1 = (NUM_ROWS + _CB1 - 1) // _CB1


def _absmax_body(wt_ref, m_ref):
    i = pl.program_id(0)
    x = wt_ref[...]

    @pl.when(i < _NBLK1 - 1)
    def _():
        m = jnp.max(jnp.abs(x))

        @pl.when(i == 0)
        def _():
            m_ref[0, 0] = m

        @pl.when(i != 0)
        def _():
            m_ref[0, 0] = jnp.maximum(m_ref[0, 0], m)

    @pl.when(i == _NBLK1 - 1)
    def _():
        col = lax.broadcasted_iota(jnp.int32, (DIM, _CB1), 1) + i * _CB1
        m = jnp.max(jnp.where(col < NUM_ROWS, jnp.abs(x), 0.0))
        m_ref[0, 0] = jnp.maximum(m_ref[0, 0], m)


def _table_absmax(Wt):
    return pl.pallas_call(
        _absmax_body,
        grid=(_NBLK1,),
        in_specs=[pl.BlockSpec((DIM, _CB1), lambda i: (0, i))],
        out_specs=pl.BlockSpec(memory_space=pltpu.SMEM),
        out_shape=jax.ShapeDtypeStruct((1, 1), jnp.float32),
    )(Wt)


def _quant_body(c0_ref, wt_ref, q_ref):
    x = wt_ref[...]
    t = jnp.tanh(x)
    qf = jnp.round(t * c0_ref[0, 0] + 127.5)
    q = qf.astype(jnp.int32)
    qp = (
        q[0:16, :]
        | (q[16:32, :] << 8)
        | (q[32:48, :] << 16)
        | (q[48:64, :] << 24)
    )
    qp2 = jnp.concatenate([qp[:, s * _CP : (s + 1) * _CP] for s in range(8)], axis=0)
    q_ref[...] = qp2.T


def _quant_table(Wt, c0):
    return pl.pallas_call(
        _quant_body,
        grid=(_NBLK,),
        in_specs=[
            pl.BlockSpec(memory_space=pltpu.SMEM),
            pl.BlockSpec((DIM, _CB), lambda i: (0, i)),
        ],
        out_specs=pl.BlockSpec((_CP, 128), lambda i: (i, 0)),
        out_shape=jax.ShapeDtypeStruct((_NBLK * _CP, 128), jnp.int32),
    )(c0, Wt)


_NC = 2
_NS = 16
_NW = _NC * _NS
_BAGS_PER_W = BATCH // _NW
_NB = DIM // 16


_BAGS_PER_CHUNK = 2
_CHUNK_ROWS = _BAGS_PER_CHUNK * HIST
_CHUNKS_PER_W = _BAGS_PER_W // _BAGS_PER_CHUNK


def _sc_body(tbl_hbm, idx_hbm, out_hbm, idx_v, rows0_v, rows1_v, out_v, sem0, sem1):
    wid = lax.axis_index("s") * _NC + lax.axis_index("c")
    pltpu.sync_copy(idx_hbm.at[pl.ds(wid * _CHUNKS_PER_W, _CHUNKS_PER_W)], idx_v)

    def process(jc, rows_v):
        for h in range(_BAGS_PER_CHUNK):
            accs = [jnp.zeros((16,), jnp.int32) for _ in range(_NB)]
            for r in range(HIST):
                packed = rows_v[h * HIST + r, :]
                accs[0] = accs[0] + (packed & 255)
                accs[1] = accs[1] + ((packed >> 8) & 255)
                accs[2] = accs[2] + ((packed >> 16) & 255)
                accs[3] = accs[3] + ((packed >> 24) & 255)
            for b in range(_NB):
                out_v[_BAGS_PER_CHUNK * jc + h, pl.ds(b * 16, 16)] = (
                    accs[b].astype(jnp.float32) * (2.0 / QSCALE) - float(HIST)
                )

    pltpu.async_copy(tbl_hbm.at[idx_v.at[0]], rows0_v, sem0)

    def pair_body(jj, carry):
        j0 = 2 * jj
        pltpu.async_copy(tbl_hbm.at[idx_v.at[j0 + 1]], rows1_v, sem1)
        pltpu.make_async_copy(tbl_hbm.at[idx_v.at[j0]], rows0_v, sem0).wait()
        process(j0, rows0_v)
        jn = jnp.minimum(j0 + 2, _CHUNKS_PER_W - 1)
        pltpu.async_copy(tbl_hbm.at[idx_v.at[jn]], rows0_v, sem0)
        pltpu.make_async_copy(tbl_hbm.at[idx_v.at[j0 + 1]], rows1_v, sem1).wait()
        process(j0 + 1, rows1_v)
        return carry

    lax.fori_loop(0, _CHUNKS_PER_W // 2, pair_body, 0)
    pltpu.make_async_copy(
        tbl_hbm.at[idx_v.at[_CHUNKS_PER_W - 1]], rows0_v, sem0
    ).wait()
    pltpu.sync_copy(out_v, out_hbm.at[pl.ds(wid * _BAGS_PER_W, _BAGS_PER_W)])


_sc_embedding_bag = functools.partial(
    pl.kernel,
    out_type=jax.ShapeDtypeStruct((BATCH, DIM), jnp.float32),
    mesh=plsc.VectorSubcoreMesh(
        core_axis_name="c", subcore_axis_name="s", num_cores=_NC, num_subcores=_NS
    ),
    scratch_types=[
        pltpu.VMEM((_CHUNKS_PER_W, _CHUNK_ROWS), jnp.int32),
        pltpu.VMEM((_CHUNK_ROWS, DIM // 4), jnp.int32),
        pltpu.VMEM((_CHUNK_ROWS, DIM // 4), jnp.int32),
        pltpu.VMEM((_BAGS_PER_W, DIM), jnp.float32),
        pltpu.SemaphoreType.DMA,
        pltpu.SemaphoreType.DMA,
    ],
    compiler_params=pltpu.CompilerParams(use_tc_tiling_on_sc=False),
)(_sc_body)

def kernel(input, W):
    Wt = W.T
    M = _table_absmax(Wt)
    c0 = QSCALE / (2.0 * jnp.tanh(M))
    tbl = _quant_table(Wt, c0).reshape(_NBLK * _CB, DIM // 4)
    i = input.astype(jnp.int32)
    rem = i % _CB
    ridx = 8 * (_CP * (i // _CB) + rem % _CP) + rem // _CP
    return _sc_embedding_bag(tbl, ridx.reshape(BATCH // _BAGS_PER_CHUNK, _CHUNK_ROWS))


# speedup vs baseline: 1.0084x; 1.0084x over previous
import functools

import jax
import jax.numpy as jnp
from jax import lax
from jax.experimental import pallas as pl
from jax.experimental.pallas import tpu as pltpu
from jax.experimental.pallas import tpu_sc as plsc

NUM_ROWS = 1_000_000
DIM = 64
HIST = 50
BATCH = 4096
QSCALE = 255.0


_CB = 32768
_NBLK = (NUM_ROWS + _CB - 1) // _CB
_CP = _CB // 8


_CB1 = 32768
_NBLK1 = (NUM_ROWS + _CB1 - 1) // _CB1


def _absmax_body(wt_ref, m_ref):
    i = pl.program_id(0)
    x = wt_ref[...]

    @pl.when(i < _NBLK1 - 1)
    def _():
        m = jnp.max(jnp.abs(x))

        @pl.when(i == 0)
        def _():
            m_ref[0, 0] = m

        @pl.when(i != 0)
        def _():
            m_ref[0, 0] = jnp.maximum(m_ref[0, 0], m)

    @pl.when(i == _NBLK1 - 1)
    def _():
        col = lax.broadcasted_iota(jnp.int32, (DIM, _CB1), 1) + i * _CB1
        m = jnp.max(jnp.where(col < NUM_ROWS, jnp.abs(x), 0.0))
        m_ref[0, 0] = jnp.maximum(m_ref[0, 0], m)


def _table_absmax(Wt):
    return pl.pallas_call(
        _absmax_body,
        grid=(_NBLK1,),
        in_specs=[pl.BlockSpec((DIM, _CB1), lambda i: (0, i))],
        out_specs=pl.BlockSpec(memory_space=pltpu.SMEM),
        out_shape=jax.ShapeDtypeStruct((1, 1), jnp.float32),
    )(Wt)


def _quant_body(c0_ref, wt_ref, q_ref):
    x = wt_ref[...]
    t = jnp.tanh(x)
    qf = jnp.round(t * c0_ref[0, 0] + 127.5)
    q = qf.astype(jnp.int32)
    qp = (
        q[0:16, :]
        | (q[16:32, :] << 8)
        | (q[32:48, :] << 16)
        | (q[48:64, :] << 24)
    )
    qp2 = jnp.concatenate([qp[:, s * _CP : (s + 1) * _CP] for s in range(8)], axis=0)
    q_ref[...] = qp2.T


def _quant_table(Wt, c0):
    return pl.pallas_call(
        _quant_body,
        grid=(_NBLK,),
        in_specs=[
            pl.BlockSpec(memory_space=pltpu.SMEM),
            pl.BlockSpec((DIM, _CB), lambda i: (0, i)),
        ],
        out_specs=pl.BlockSpec((_CP, 128), lambda i: (i, 0)),
        out_shape=jax.ShapeDtypeStruct((_NBLK * _CP, 128), jnp.int32),
    )(c0, Wt)


_NC = 2
_NS = 16
_NW = _NC * _NS
_BAGS_PER_W = BATCH // _NW
_NB = DIM // 16


_BAGS_PER_CHUNK = 2
_CHUNK_ROWS = _BAGS_PER_CHUNK * HIST
_CHUNKS_PER_W = _BAGS_PER_W // _BAGS_PER_CHUNK


def _sc_body(tbl_hbm, idx_hbm, out_hbm, idx_v, rows0_v, rows1_v, out_v, sem0, sem1):
    wid = lax.axis_index("s") * _NC + lax.axis_index("c")
    pltpu.sync_copy(idx_hbm.at[pl.ds(wid * _CHUNKS_PER_W, _CHUNKS_PER_W)], idx_v)

    def process(jc, rows_v):
        for h in range(_BAGS_PER_CHUNK):
            accs = [jnp.zeros((16,), jnp.int32) for _ in range(_NB)]
            for r in range(HIST):
                packed = rows_v[h * HIST + r, :]
                accs[0] = accs[0] + (packed & 255)
                accs[1] = accs[1] + ((packed >> 8) & 255)
                accs[2] = accs[2] + ((packed >> 16) & 255)
                accs[3] = accs[3] + ((packed >> 24) & 255)
            for b in range(_NB):
                out_v[_BAGS_PER_CHUNK * jc + h, pl.ds(b * 16, 16)] = (
                    accs[b].astype(jnp.float32) * (2.0 / QSCALE) - float(HIST)
                )

    pltpu.async_copy(tbl_hbm.at[idx_v.at[0]], rows0_v, sem0)

    def pair_body(jj, carry):
        j0 = 2 * jj
        pltpu.async_copy(tbl_hbm.at[idx_v.at[j0 + 1]], rows1_v, sem1)
        pltpu.make_async_copy(tbl_hbm.at[idx_v.at[j0]], rows0_v, sem0).wait()
        process(j0, rows0_v)
        jn = jnp.minimum(j0 + 2, _CHUNKS_PER_W - 1)
        pltpu.async_copy(tbl_hbm.at[idx_v.at[jn]], rows0_v, sem0)
        pltpu.make_async_copy(tbl_hbm.at[idx_v.at[j0 + 1]], rows1_v, sem1).wait()
        process(j0 + 1, rows1_v)
        return carry

    lax.fori_loop(0, _CHUNKS_PER_W // 2, pair_body, 0)
    pltpu.make_async_copy(
        tbl_hbm.at[idx_v.at[_CHUNKS_PER_W - 1]], rows0_v, sem0
    ).wait()
    pltpu.sync_copy(out_v, out_hbm.at[pl.ds(wid * _BAGS_PER_W, _BAGS_PER_W)])


_sc_embedding_bag = functools.partial(
    pl.kernel,
    out_type=jax.ShapeDtypeStruct((BATCH, DIM), jnp.float32),
    mesh=plsc.VectorSubcoreMesh(
        core_axis_name="c", subcore_axis_name="s", num_cores=_NC, num_subcores=_NS
    ),
    scratch_types=[
        pltpu.VMEM((_CHUNKS_PER_W, _CHUNK_ROWS), jnp.int32),
        pltpu.VMEM((_CHUNK_ROWS, DIM // 4), jnp.int32),
        pltpu.VMEM((_CHUNK_ROWS, DIM // 4), jnp.int32),
        pltpu.VMEM((_BAGS_PER_W, DIM), jnp.float32),
        pltpu.SemaphoreType.DMA,
        pltpu.SemaphoreType.DMA,
    ],
    compiler_params=pltpu.CompilerParams(use_tc_tiling_on_sc=False),
)(_sc_body)

def kernel(input, W):
    Wt = W.T
    M = _table_absmax(Wt)
    c0 = QSCALE / (2.0 * jnp.tanh(M))
    tbl = _quant_table(Wt, c0).reshape(_NBLK * _CB, DIM // 4)
    i = input.astype(jnp.int32)
    rem = i % _CB
    ridx = 8 * (_CP * (i // _CB) + rem % _CP) + rem // _CP
    return _sc_embedding_bag(tbl, ridx.reshape(BATCH // _BAGS_PER_CHUNK, _CHUNK_ROWS))
